# baseline (speedup 1.0000x reference)
import functools

import jax
import jax.numpy as jnp
from jax import lax
from jax.experimental import pallas as pl
from jax.experimental.pallas import tpu as pltpu
from jax.experimental.pallas import tpu_sc as plsc

EMB = 128
TIME = 128
EDGE = 16
KN = 16
H = 8
QD = EMB + TIME
KD = EMB + EDGE + TIME
KDIM = KD * KN
N = 2048
DH = QD // H

NC = 2
NS = 16
NW = NC * NS
NPW = N // NW
NBUF = 12


def _sc_gather(features, idxn2d, idxt3d):
  mesh = plsc.VectorSubcoreMesh(core_axis_name="c", subcore_axis_name="s")

  @functools.partial(
      pl.kernel,
      mesh=mesh,
      out_type=[
          jax.ShapeDtypeStruct((N, KN * EMB), jnp.float32),
          jax.ShapeDtypeStruct((N, EMB), jnp.float32),
      ],
      compiler_params=pltpu.CompilerParams(use_tc_tiling_on_sc=True),
      scratch_types=[
          pltpu.VMEM((KN, NPW), jnp.int32),
          pltpu.VMEM((NPW,), jnp.int32),
          pltpu.VMEM((NPW, EMB), jnp.float32),
      ] + [pltpu.VMEM((NPW, EMB), jnp.float32)] * NBUF
        + [pltpu.SemaphoreType.DMA] * (2 * NBUF + 1),
  )
  def gather_kernel(table_hbm, idxn_hbm, idx_hbm, out_nbr, out_node,
                    idx_v, idxn_v, nbuf, *rest):
    bufs = rest[:NBUF]
    gsems = rest[NBUF:2 * NBUF]
    osems = rest[2 * NBUF:3 * NBUF]
    nsem = rest[3 * NBUF]
    wid = lax.axis_index("s") * NC + lax.axis_index("c")
    pltpu.sync_copy(idx_hbm.at[wid], idx_v)
    pltpu.sync_copy(idxn_hbm.at[wid], idxn_v)
    ncp = pltpu.async_copy(table_hbm.at[idxn_v], nbuf, nsem)
    G = [None] * KN
    O = [None] * KN
    for j in range(min(NBUF, KN)):
      G[j] = pltpu.async_copy(table_hbm.at[idx_v.at[j]], bufs[j], gsems[j])
    for j in range(KN):
      i = j % NBUF
      G[j].wait()
      O[j] = pltpu.async_copy(
          bufs[i],
          out_nbr.at[pl.ds(wid * NPW, NPW), pl.ds(j * EMB, EMB)],
          osems[i])
      nxt = j + NBUF
      if nxt < KN:
        O[j].wait()
        G[nxt] = pltpu.async_copy(table_hbm.at[idx_v.at[nxt]], bufs[i],
                                  gsems[i])
    ncp.wait()
    pltpu.sync_copy(nbuf, out_node.at[pl.ds(wid * NPW, NPW)])
    for j in range(max(KN - NBUF, 0), KN):
      O[j].wait()

  return gather_kernel(features, idxn2d, idxt3d)


def _dot(a, b):
  return lax.dot_general(a, b, (((1,), (0,)), ((), ())),
                         preferred_element_type=jnp.float32)


def _dot_t(a, b):
  return lax.dot_general(a, b, (((1,), (1,)), ((), ())),
                         preferred_element_type=jnp.float32)


def _regroup_body(kw_ref, vw_ref, ak_ref, bk_ref, ck_ref,
                  av_ref, bv_ref, cv_ref):
  kw = kw_ref[...]
  vw = vw_ref[...]
  for w, a_r, b_r, c_r in ((kw, ak_ref, bk_ref, ck_ref),
                           (vw, av_ref, bv_ref, cv_ref)):
    for j in range(KN):
      base = j * KD
      a_r[:, j * EMB:(j + 1) * EMB] = w[:, base:base + EMB]
      b_r[:, j * EDGE:(j + 1) * EDGE] = w[:, base + EMB:base + EMB + EDGE]
      c_r[:, j * TIME:(j + 1) * TIME] = w[:, base + EMB + EDGE:base + KD]


def _regroup(k_w, v_w, interpret=False):
  one = lambda shape: pl.BlockSpec(shape, lambda: (0, 0))
  big = jax.ShapeDtypeStruct((QD, KN * EMB), jnp.float32)
  sml = jax.ShapeDtypeStruct((QD, KN * EDGE), jnp.float32)
  return pl.pallas_call(
      _regroup_body,
      in_specs=[one((QD, KDIM)), one((QD, KDIM))],
      out_specs=[one((QD, KN * EMB)), one((QD, KN * EDGE)),
                 one((QD, KN * TIME))] * 2,
      out_shape=[big, sml, big, big, sml, big],
      interpret=interpret,
  )(k_w, v_w)


def _proj_et_body(edge_ref, time_ref, bk_ref, ck_ref, bv_ref, cv_ref,
                  inb_ref, kp_ref, vp_ref):
  kp_ref[...] = (_dot_t(edge_ref[...], bk_ref[...]) +
                 _dot_t(time_ref[...], ck_ref[...]) + inb_ref[1:2, :])
  vp_ref[...] = (_dot_t(edge_ref[...], bv_ref[...]) +
                 _dot_t(time_ref[...], cv_ref[...]) + inb_ref[2:3, :])


def _proj_nbr_body(node_ref, nbr_ref, qw_ref, ak_ref, av_ref, inb_ref,
                   kp_ref, vp_ref, q_ref, k_ref, v_ref):
  q_ref[...] = _dot_t(node_ref[...], qw_ref[...]) + inb_ref[0:1, :]
  k_ref[...] = _dot_t(nbr_ref[...], ak_ref[...]) + kp_ref[...]
  v_ref[...] = _dot_t(nbr_ref[...], av_ref[...]) + vp_ref[...]


def _attn_body(q_ref, k_ref, v_ref, node_ref, outw_ref, outb_ref,
               w1n_ref, w1a_ref, b1_ref, w2_ref, b2_ref, o_ref, ctx_ref):
  scale2 = float(1.4426950408889634 / (DH ** 0.5))
  q = (q_ref[...] * scale2).astype(jnp.bfloat16)
  k = k_ref[...].astype(jnp.bfloat16)
  v = v_ref[...].astype(jnp.bfloat16)
  ones = jnp.ones((N, 1), jnp.bfloat16)
  for h in range(H):
    sl = slice(h * DH, (h + 1) * DH)
    s = _dot_t(q[:, sl], k[:, sl])
    eb = jnp.exp2(s).astype(jnp.bfloat16)
    u = _dot(eb, jnp.concatenate([v[:, sl], ones], axis=1))
    ctx_ref[:, sl] = u[:, :DH] * (1.0 / u[:, DH:DH + 1])
  attn = _dot_t(ctx_ref[...], outw_ref[...]) + outb_ref[...]
  h1 = jnp.maximum(
      _dot_t(node_ref[...], w1n_ref[...]) + _dot_t(attn, w1a_ref[...])
      + b1_ref[...], 0.0)
  o_ref[...] = _dot_t(h1, w2_ref[...]) + b2_ref[...]


def _proj_et(edge_flat, time_flat, bk, ck, bv, cv, inb3, interpret=False):
  BN = 256
  full = lambda shape: pl.BlockSpec(shape, lambda i: (0, 0))
  row = lambda shape: pl.BlockSpec(shape, lambda i: (i, 0))
  return pl.pallas_call(
      _proj_et_body,
      grid=(N // BN,),
      in_specs=[
          row((BN, KN * EDGE)), row((BN, KN * TIME)),
          full((QD, KN * EDGE)), full((QD, KN * TIME)),
          full((QD, KN * EDGE)), full((QD, KN * TIME)), full((8, QD)),
      ],
      out_specs=[row((BN, QD)), row((BN, QD))],
      out_shape=[jax.ShapeDtypeStruct((N, QD), jnp.float32)] * 2,
      interpret=interpret,
  )(edge_flat, time_flat, bk, ck, bv, cv, inb3)


def _dense(node_emb, nbr_flat, kpart, vpart, qw_e,
           ak, av, inb3, outw, outb2, w1n, w1a, b12,
           w2, b22, interpret=False):
  BN = 256
  full = lambda shape: pl.BlockSpec(shape, lambda i: (0, 0))
  row = lambda shape: pl.BlockSpec(shape, lambda i: (i, 0))
  q, k, v = pl.pallas_call(
      _proj_nbr_body,
      grid=(N // BN,),
      in_specs=[
          row((BN, EMB)), row((BN, KN * EMB)),
          full((QD, EMB)), full((QD, KN * EMB)), full((QD, KN * EMB)),
          full((8, QD)), row((BN, QD)), row((BN, QD)),
      ],
      out_specs=[row((BN, QD)), row((BN, QD)), row((BN, QD))],
      out_shape=[jax.ShapeDtypeStruct((N, QD), jnp.float32)] * 3,
      interpret=interpret,
  )(node_emb, nbr_flat, qw_e, ak, av, inb3, kpart, vpart)

  BQ = 1024
  out = pl.pallas_call(
      _attn_body,
      grid=(N // BQ,),
      in_specs=[
          pl.BlockSpec((BQ, QD), lambda i: (i, 0)),
          pl.BlockSpec((N, QD), lambda i: (0, 0)),
          pl.BlockSpec((N, QD), lambda i: (0, 0)),
          pl.BlockSpec((BQ, EMB), lambda i: (i, 0)),
          full((QD, QD)), full((1, QD)),
          full((EMB, EMB)), full((EMB, QD)), full((1, EMB)),
          full((EMB, EMB)), full((1, EMB)),
      ],
      out_specs=pl.BlockSpec((BQ, EMB), lambda i: (i, 0)),
      out_shape=jax.ShapeDtypeStruct((N, EMB), jnp.float32),
      scratch_shapes=[pltpu.VMEM((BQ, QD), jnp.float32)],
      interpret=interpret,
  )(q, k, v, node_emb, outw, outb2, w1n, w1a, b12, w2, b22)
  return out


def kernel(features, edge_feats, time_feats, q_w, k_w, v_w, in_b, out_w,
           out_b, w1, b1, w2, b2, neighbor_idx, node_idx):
  n = node_idx.shape[0]
  idxt3d = neighbor_idx.astype(jnp.int32).reshape(NW, NPW, KN).transpose(0, 2, 1)
  idxn2d = node_idx.astype(jnp.int32).reshape(NW, NPW)
  nbr_flat, node_emb = _sc_gather(features, idxn2d, idxt3d)

  ak, bk, ck, av, bv, cv = _regroup(k_w, v_w)
  qw_e = q_w[:, :EMB]
  inb3 = jnp.zeros((8, QD), jnp.float32).at[:3].set(in_b.reshape(3, QD))
  edge_flat = edge_feats.reshape(n, KN * EDGE)
  time_flat = time_feats.reshape(n, KN * TIME)
  kpart, vpart = _proj_et(edge_flat, time_flat, bk, ck, bv, cv, inb3)

  return _dense(node_emb, nbr_flat, kpart, vpart, qw_e,
                ak, av, inb3,
                out_w, out_b.reshape(1, QD), w1[:, :EMB], w1[:, EMB:],
                b1.reshape(1, EMB), w2, b2.reshape(1, EMB))


# speedup vs baseline: 1.0067x; 1.0067x over previous
import functools

import jax
import jax.numpy as jnp
from jax import lax
from jax.experimental import pallas as pl
from jax.experimental.pallas import tpu as pltpu
from jax.experimental.pallas import tpu_sc as plsc

EMB = 128
TIME = 128
EDGE = 16
KN = 16
H = 8
QD = EMB + TIME
KD = EMB + EDGE + TIME
KDIM = KD * KN
N = 2048
DH = QD // H

NC = 2
NS = 16
NW = NC * NS
NPW = N // NW
NBUF = 12


def _sc_gather(features, idxn2d, idxt3d):
  mesh = plsc.VectorSubcoreMesh(core_axis_name="c", subcore_axis_name="s")

  @functools.partial(
      pl.kernel,
      mesh=mesh,
      out_type=[
          jax.ShapeDtypeStruct((N, KN * EMB), jnp.float32),
          jax.ShapeDtypeStruct((N, EMB), jnp.float32),
      ],
      compiler_params=pltpu.CompilerParams(use_tc_tiling_on_sc=True),
      scratch_types=[
          pltpu.VMEM((KN, NPW), jnp.int32),
          pltpu.VMEM((NPW,), jnp.int32),
          pltpu.VMEM((NPW, EMB), jnp.float32),
      ] + [pltpu.VMEM((NPW, EMB), jnp.float32)] * NBUF
        + [pltpu.SemaphoreType.DMA] * (2 * NBUF + 1),
  )
  def gather_kernel(table_hbm, idxn_hbm, idx_hbm, out_nbr, out_node,
                    idx_v, idxn_v, nbuf, *rest):
    bufs = rest[:NBUF]
    gsems = rest[NBUF:2 * NBUF]
    osems = rest[2 * NBUF:3 * NBUF]
    nsem = rest[3 * NBUF]
    wid = lax.axis_index("s") * NC + lax.axis_index("c")
    pltpu.sync_copy(idx_hbm.at[wid], idx_v)
    pltpu.sync_copy(idxn_hbm.at[wid], idxn_v)
    ncp = pltpu.async_copy(table_hbm.at[idxn_v], nbuf, nsem)
    G = [None] * KN
    O = [None] * KN
    for j in range(min(NBUF, KN)):
      G[j] = pltpu.async_copy(table_hbm.at[idx_v.at[j]], bufs[j], gsems[j])
    for j in range(KN):
      i = j % NBUF
      G[j].wait()
      O[j] = pltpu.async_copy(
          bufs[i],
          out_nbr.at[pl.ds(wid * NPW, NPW), pl.ds(j * EMB, EMB)],
          osems[i])
      nxt = j + NBUF
      if nxt < KN:
        O[j].wait()
        G[nxt] = pltpu.async_copy(table_hbm.at[idx_v.at[nxt]], bufs[i],
                                  gsems[i])
    ncp.wait()
    pltpu.sync_copy(nbuf, out_node.at[pl.ds(wid * NPW, NPW)])
    for j in range(max(KN - NBUF, 0), KN):
      O[j].wait()

  return gather_kernel(features, idxn2d, idxt3d)


def _dot(a, b):
  return lax.dot_general(a, b, (((1,), (0,)), ((), ())),
                         preferred_element_type=jnp.float32)


def _dot_t(a, b):
  return lax.dot_general(a, b, (((1,), (1,)), ((), ())),
                         preferred_element_type=jnp.float32)


def _regroup_body(kw_ref, vw_ref, ak_ref, bk_ref, ck_ref,
                  av_ref, bv_ref, cv_ref):
  kw = kw_ref[...]
  vw = vw_ref[...]
  for w, a_r, b_r, c_r in ((kw, ak_ref, bk_ref, ck_ref),
                           (vw, av_ref, bv_ref, cv_ref)):
    for j in range(KN):
      base = j * KD
      a_r[:, j * EMB:(j + 1) * EMB] = w[:, base:base + EMB]
      b_r[:, j * EDGE:(j + 1) * EDGE] = w[:, base + EMB:base + EMB + EDGE]
      c_r[:, j * TIME:(j + 1) * TIME] = w[:, base + EMB + EDGE:base + KD]


def _regroup(k_w, v_w, interpret=False):
  one = lambda shape: pl.BlockSpec(shape, lambda: (0, 0))
  big = jax.ShapeDtypeStruct((QD, KN * EMB), jnp.float32)
  sml = jax.ShapeDtypeStruct((QD, KN * EDGE), jnp.float32)
  return pl.pallas_call(
      _regroup_body,
      in_specs=[one((QD, KDIM)), one((QD, KDIM))],
      out_specs=[one((QD, KN * EMB)), one((QD, KN * EDGE)),
                 one((QD, KN * TIME))] * 2,
      out_shape=[big, sml, big, big, sml, big],
      interpret=interpret,
  )(k_w, v_w)


def _proj_et_body(edge_ref, time_ref, bk_ref, ck_ref, bv_ref, cv_ref,
                  inb_ref, kp_ref, vp_ref):
  kp_ref[...] = (_dot_t(edge_ref[...], bk_ref[...]) +
                 _dot_t(time_ref[...], ck_ref[...]) + inb_ref[1:2, :])
  vp_ref[...] = (_dot_t(edge_ref[...], bv_ref[...]) +
                 _dot_t(time_ref[...], cv_ref[...]) + inb_ref[2:3, :])


def _proj_nbr_body(node_ref, nbr_ref, qw_ref, ak_ref, av_ref, inb_ref,
                   kp_ref, vp_ref, q_ref, k_ref, v_ref):
  q_ref[...] = _dot_t(node_ref[...], qw_ref[...]) + inb_ref[0:1, :]
  k_ref[...] = _dot_t(nbr_ref[...], ak_ref[...]) + kp_ref[...]
  v_ref[...] = _dot_t(nbr_ref[...], av_ref[...]) + vp_ref[...]


def _attn_body(q_ref, k_ref, v_ref, node_ref, outw_ref, outb_ref,
               w1n_ref, w1a_ref, b1_ref, w2_ref, b2_ref, o_ref, ctx_ref):
  scale2 = float(1.4426950408889634 / (DH ** 0.5))
  q = q_ref[...] * scale2
  k = k_ref[...]
  v = v_ref[...]
  ones = jnp.ones((N, 1), jnp.float32)
  for h in range(H):
    sl = slice(h * DH, (h + 1) * DH)
    s = _dot_t(q[:, sl], k[:, sl])
    e = jnp.exp2(s)
    u = _dot(e, jnp.concatenate([v[:, sl], ones], axis=1))
    ctx_ref[:, sl] = u[:, :DH] * (1.0 / u[:, DH:DH + 1])
  attn = _dot_t(ctx_ref[...], outw_ref[...]) + outb_ref[...]
  h1 = jnp.maximum(
      _dot_t(node_ref[...], w1n_ref[...]) + _dot_t(attn, w1a_ref[...])
      + b1_ref[...], 0.0)
  o_ref[...] = _dot_t(h1, w2_ref[...]) + b2_ref[...]


def _proj_et(edge_flat, time_flat, bk, ck, bv, cv, inb3, interpret=False):
  BN = 256
  full = lambda shape: pl.BlockSpec(shape, lambda i: (0, 0))
  row = lambda shape: pl.BlockSpec(shape, lambda i: (i, 0))
  return pl.pallas_call(
      _proj_et_body,
      grid=(N // BN,),
      in_specs=[
          row((BN, KN * EDGE)), row((BN, KN * TIME)),
          full((QD, KN * EDGE)), full((QD, KN * TIME)),
          full((QD, KN * EDGE)), full((QD, KN * TIME)), full((8, QD)),
      ],
      out_specs=[row((BN, QD)), row((BN, QD))],
      out_shape=[jax.ShapeDtypeStruct((N, QD), jnp.float32)] * 2,
      interpret=interpret,
  )(edge_flat, time_flat, bk, ck, bv, cv, inb3)


def _dense(node_emb, nbr_flat, kpart, vpart, qw_e,
           ak, av, inb3, outw, outb2, w1n, w1a, b12,
           w2, b22, interpret=False):
  BN = 256
  full = lambda shape: pl.BlockSpec(shape, lambda i: (0, 0))
  row = lambda shape: pl.BlockSpec(shape, lambda i: (i, 0))
  q, k, v = pl.pallas_call(
      _proj_nbr_body,
      grid=(N // BN,),
      in_specs=[
          row((BN, EMB)), row((BN, KN * EMB)),
          full((QD, EMB)), full((QD, KN * EMB)), full((QD, KN * EMB)),
          full((8, QD)), row((BN, QD)), row((BN, QD)),
      ],
      out_specs=[row((BN, QD)), row((BN, QD)), row((BN, QD))],
      out_shape=[jax.ShapeDtypeStruct((N, QD), jnp.float32)] * 3,
      interpret=interpret,
  )(node_emb, nbr_flat, qw_e, ak, av, inb3, kpart, vpart)

  BQ = 1024
  out = pl.pallas_call(
      _attn_body,
      grid=(N // BQ,),
      in_specs=[
          pl.BlockSpec((BQ, QD), lambda i: (i, 0)),
          pl.BlockSpec((N, QD), lambda i: (0, 0)),
          pl.BlockSpec((N, QD), lambda i: (0, 0)),
          pl.BlockSpec((BQ, EMB), lambda i: (i, 0)),
          full((QD, QD)), full((1, QD)),
          full((EMB, EMB)), full((EMB, QD)), full((1, EMB)),
          full((EMB, EMB)), full((1, EMB)),
      ],
      out_specs=pl.BlockSpec((BQ, EMB), lambda i: (i, 0)),
      out_shape=jax.ShapeDtypeStruct((N, EMB), jnp.float32),
      scratch_shapes=[pltpu.VMEM((BQ, QD), jnp.float32)],
      interpret=interpret,
  )(q, k, v, node_emb, outw, outb2, w1n, w1a, b12, w2, b22)
  return out


def kernel(features, edge_feats, time_feats, q_w, k_w, v_w, in_b, out_w,
           out_b, w1, b1, w2, b2, neighbor_idx, node_idx):
  n = node_idx.shape[0]
  idxt3d = neighbor_idx.astype(jnp.int32).reshape(NW, NPW, KN).transpose(0, 2, 1)
  idxn2d = node_idx.astype(jnp.int32).reshape(NW, NPW)
  nbr_flat, node_emb = _sc_gather(features, idxn2d, idxt3d)

  ak, bk, ck, av, bv, cv = _regroup(k_w, v_w)
  qw_e = q_w[:, :EMB]
  inb3 = jnp.zeros((8, QD), jnp.float32).at[:3].set(in_b.reshape(3, QD))
  edge_flat = edge_feats.reshape(n, KN * EDGE)
  time_flat = time_feats.reshape(n, KN * TIME)
  kpart, vpart = _proj_et(edge_flat, time_flat, bk, ck, bv, cv, inb3)

  return _dense(node_emb, nbr_flat, kpart, vpart, qw_e,
                ak, av, inb3,
                out_w, out_b.reshape(1, QD), w1[:, :EMB], w1[:, EMB:],
                b1.reshape(1, EMB), w2, b2.reshape(1, EMB))
